# baseline (speedup 1.0000x reference)
import functools

import jax
import jax.numpy as jnp
from jax import lax
from jax.experimental import pallas as pl
from jax.experimental.pallas import tpu as pltpu
from jax.experimental.pallas import tpu_sc as plsc

_NC, _NS, _LANES = 2, 16, 16
_NW = _NC * _NS

_D = 64


def _fuse_body(seq_len, table_ref, pw_ref, pb_ref, out_ref):
    blk = table_ref[...]
    vvec = pw_ref[2:3, :] + pw_ref[3:4, :]
    pb = pb_ref[...]
    blk_v = blk.shape[0]
    row = lax.broadcasted_iota(jnp.int32, (blk_v, 1), 0)
    wcol = ((row % 8) + 1).astype(jnp.float32) * (1.0 / seq_len)
    out_ref[...] = blk + wcol * vvec + pb


def _fuse_table(table, pos_W, pos_b, seq_len):
    vocab, d = table.shape
    blk_v = 4000
    grid = vocab // blk_v
    return pl.pallas_call(
        functools.partial(_fuse_body, seq_len),
        grid=(grid,),
        in_specs=[
            pl.BlockSpec((blk_v, d), lambda i: (i, 0)),
            pl.BlockSpec((4, d), lambda i: (0, 0)),
            pl.BlockSpec((1, d), lambda i: (0, 0)),
        ],
        out_specs=pl.BlockSpec((blk_v, d), lambda i: (i, 0)),
        out_shape=jax.ShapeDtypeStruct((vocab, d), jnp.float32),
    )(table, pos_W, pos_b.reshape(1, d))


def _xs_body(ids_ref, out_ref):
    ids = ids_ref[...]
    seq = ids.shape[1]
    tl = ((ids % 8) + 1).astype(jnp.float32)
    r = lax.broadcasted_iota(jnp.int32, (seq, seq), 0)
    c = lax.broadcasted_iota(jnp.int32, (seq, seq), 1)
    tri = (r < c).astype(jnp.float32)
    out_ref[...] = jnp.dot(
        tl, tri,
        preferred_element_type=jnp.float32,
        precision=lax.Precision.HIGHEST,
    ) * (1.0 / seq)


def _xs_compute(ids):
    b, seq = ids.shape
    blk_b = 512
    grid = b // blk_b
    return pl.pallas_call(
        _xs_body,
        grid=(grid,),
        in_specs=[pl.BlockSpec((blk_b, seq), lambda i: (i, 0))],
        out_specs=pl.BlockSpec((blk_b, seq), lambda i: (i, 0)),
        out_shape=jax.ShapeDtypeStruct((b, seq), jnp.float32),
    )(ids)


def _make_sc_embed(batch, seq, nbuf=4, unroll=8):
    chunk = seq
    nchunk = batch // _NW
    ngroup = nchunk // nbuf
    mesh = plsc.VectorSubcoreMesh(core_axis_name="c", subcore_axis_name="s")

    scratch = (
        [pltpu.VMEM((chunk,), jnp.int32) for _ in range(nbuf)]
        + [pltpu.VMEM((chunk,), jnp.float32) for _ in range(nbuf)]
        + [pltpu.VMEM((chunk, _D), jnp.float32) for _ in range(nbuf)]
        + [pltpu.VMEM((_D,), jnp.float32)]
        + [pltpu.SemaphoreType.DMA for _ in range(2 * nbuf)]
    )

    @functools.partial(
        pl.kernel,
        out_type=jax.ShapeDtypeStruct((batch, seq, _D), jnp.float32),
        mesh=mesh,
        scratch_types=scratch,
        compiler_params=pltpu.CompilerParams(
            needs_layout_passes=False, use_tc_tiling_on_sc=False
        ),
    )
    def sc_embed(idx_hbm, xs_hbm, table_hbm, u_hbm, out_hbm, *scr):
        idx_v = scr[0:nbuf]
        xs_v = scr[nbuf:2 * nbuf]
        rows_v = scr[2 * nbuf:3 * nbuf]
        u_v = scr[3 * nbuf]
        gsem = scr[3 * nbuf + 1:3 * nbuf + 1 + nbuf]
        osem = scr[3 * nbuf + 1 + nbuf:3 * nbuf + 1 + 2 * nbuf]

        wid = lax.axis_index("s") * _NC + lax.axis_index("c")
        base = wid * nchunk
        pltpu.sync_copy(u_hbm, u_v)
        u_regs = [u_v[pl.ds(16 * k, 16)] for k in range(_D // 16)]

        def stage(s, slot):
            pltpu.sync_copy(idx_hbm.at[base + s], idx_v[slot])
            pltpu.async_copy(table_hbm.at[idx_v[slot]], rows_v[slot],
                             gsem[slot])
            pltpu.sync_copy(xs_hbm.at[base + s], xs_v[slot])

        def compute(slot):
            @plsc.parallel_loop(0, chunk, 1, unroll=unroll)
            def _tok(t):
                xb = plsc.load_gather(
                    xs_v[slot], [jnp.broadcast_to(t, (16,)).astype(jnp.int32)]
                )
                for k in range(_D // 16):
                    plsc.addupdate(
                        rows_v[slot].at[t, pl.ds(16 * k, 16)], xb * u_regs[k]
                    )

        ahead = 2
        for b in range(ahead):
            stage(b, b)

        def group_body(g, _):
            for b in range(nbuf):
                s = g * nbuf + b
                slot_n = (b + ahead) % nbuf

                @pl.when(s + ahead < nchunk)
                def _():
                    pltpu.sync_copy(idx_hbm.at[base + s + ahead],
                                    idx_v[slot_n])

                    @pl.when(s + ahead - nbuf >= 0)
                    def _():
                        pltpu.make_async_copy(
                            rows_v[slot_n], out_hbm.at[0],
                            osem[slot_n]).wait()

                    pltpu.async_copy(table_hbm.at[idx_v[slot_n]],
                                     rows_v[slot_n], gsem[slot_n])
                    pltpu.sync_copy(xs_hbm.at[base + s + ahead], xs_v[slot_n])

                pltpu.make_async_copy(
                    table_hbm.at[idx_v[b]], rows_v[b], gsem[b]).wait()
                compute(b)
                pltpu.async_copy(rows_v[b], out_hbm.at[base + s], osem[b])
            return 0

        lax.fori_loop(0, ngroup, group_body, 0)

        for b in range(nbuf):
            pltpu.make_async_copy(
                rows_v[b], out_hbm.at[0], osem[b]).wait()

    return sc_embed


def kernel(input_ids, token_table, pos_W, pos_b):
    b, seq = input_ids.shape
    d = token_table.shape[1]
    ids = input_ids.astype(jnp.int32)

    table_fused = _fuse_table(token_table, pos_W, pos_b, seq)
    xs = _xs_compute(ids)
    u = pos_W[0] + pos_W[1]

    del d
    return _make_sc_embed(b, seq, nbuf=4, unroll=8)(
        ids, xs, table_fused, u
    )


# speedup vs baseline: 1.0740x; 1.0740x over previous
import functools

import jax
import jax.numpy as jnp
from jax import lax
from jax.experimental import pallas as pl
from jax.experimental.pallas import tpu as pltpu
from jax.experimental.pallas import tpu_sc as plsc

_NC, _NS, _LANES = 2, 16, 16
_NW = _NC * _NS

_D = 64


def _make_sc_gather(n_tokens, chunk=128, nbuf=5):
    npw = n_tokens // _NW
    nchunk = npw // chunk
    ngroup = nchunk // nbuf
    mesh = plsc.VectorSubcoreMesh(core_axis_name="c", subcore_axis_name="s")

    panel = 3200
    scratch = (
        [pltpu.VMEM((chunk,), jnp.int32) for _ in range(nbuf)]
        + [pltpu.VMEM((chunk, _D), jnp.float32) for _ in range(nbuf)]
        + [pltpu.SemaphoreType.DMA for _ in range(2 * nbuf)]
    )

    @functools.partial(
        pl.kernel,
        out_type=jax.ShapeDtypeStruct((n_tokens // 2, 2 * _D), jnp.float32),
        mesh=mesh,
        scratch_types=scratch,
        compiler_params=pltpu.CompilerParams(
            needs_layout_passes=False, use_tc_tiling_on_sc=False
        ),
    )
    def sc_gather(idx_hbm, table_hbm, out_hbm, *scr):
        idx_v = scr[0:nbuf]
        rows_v = scr[nbuf:2 * nbuf]
        gsem = scr[2 * nbuf:3 * nbuf]
        osem = scr[3 * nbuf:4 * nbuf]

        wid = lax.axis_index("s") * _NC + lax.axis_index("c")
        base = wid * npw

        def stage(s, slot):
            pltpu.sync_copy(idx_hbm.at[pl.ds(base + s * chunk, chunk)],
                            idx_v[slot])
            pltpu.async_copy(table_hbm.at[idx_v[slot]], rows_v[slot],
                             gsem[slot])

        def out_copy(s, slot, sem):
            t0 = base + s * chunk
            rem = t0 % (2 * panel)
            odd = rem >= panel
            r0 = (t0 // (2 * panel)) * panel + rem - jnp.where(odd, panel, 0)
            lane0 = jnp.where(odd, _D, 0)
            return pltpu.make_async_copy(
                rows_v[slot],
                out_hbm.at[pl.ds(r0, chunk), pl.ds(lane0, _D)],
                sem)

        ahead = 2
        for b in range(ahead):
            stage(b, b)

        def group_body(g, _):
            for b in range(nbuf):
                s = g * nbuf + b
                slot_n = (b + ahead) % nbuf

                @pl.when(s + ahead < nchunk)
                def _():
                    pltpu.sync_copy(
                        idx_hbm.at[pl.ds(base + (s + ahead) * chunk, chunk)],
                        idx_v[slot_n])

                    @pl.when(s + ahead - nbuf >= 0)
                    def _():
                        out_copy(0, slot_n, osem[slot_n]).wait()

                    pltpu.async_copy(table_hbm.at[idx_v[slot_n]],
                                     rows_v[slot_n], gsem[slot_n])

                pltpu.make_async_copy(
                    table_hbm.at[idx_v[b]], rows_v[b], gsem[b]).wait()
                out_copy(s, b, osem[b]).start()
            return 0

        lax.fori_loop(0, ngroup, group_body, 0)

        for b in range(nbuf):
            out_copy(0, b, osem[b]).wait()

    return sc_gather


def _finalize_body(ids_ref, g2_ref, pw_ref, pb_ref, out_ref):
    bb, seq = ids_ref.shape
    x2 = g2_ref[...]
    toks = jnp.concatenate([x2[:, :_D], x2[:, _D:]], axis=0)
    toks = toks.reshape(bb, seq, _D)

    ids = ids_ref[...]
    tl = ((ids % 8) + 1).astype(jnp.float32)
    r = lax.broadcasted_iota(jnp.int32, (seq, seq), 0)
    c = lax.broadcasted_iota(jnp.int32, (seq, seq), 1)
    tri = (r < c).astype(jnp.float32)
    start = jnp.dot(tl, tri, preferred_element_type=jnp.float32,
                    precision=lax.Precision.HIGHEST)
    xs = start * (1.0 / seq)
    ws = tl * (1.0 / seq)

    u = (pw_ref[0:1, :] + pw_ref[1:2, :]).reshape(1, 1, _D)
    v = (pw_ref[2:3, :] + pw_ref[3:4, :]).reshape(1, 1, _D)
    pb = pb_ref[...].reshape(1, 1, _D)
    out_ref[...] = (toks + xs[:, :, None] * u + ws[:, :, None] * v + pb)


def _finalize(ids, g2, pos_W, pos_b):
    batch, seq = ids.shape
    bb = 32
    grid = batch // bb
    return pl.pallas_call(
        _finalize_body,
        grid=(grid,),
        in_specs=[
            pl.BlockSpec((bb, seq), lambda i: (i, 0)),
            pl.BlockSpec((bb * seq // 2, 2 * _D), lambda i: (i, 0)),
            pl.BlockSpec((4, _D), lambda i: (0, 0)),
            pl.BlockSpec((1, _D), lambda i: (0, 0)),
        ],
        out_specs=pl.BlockSpec((bb, seq, _D), lambda i: (i, 0, 0)),
        out_shape=jax.ShapeDtypeStruct((batch, seq, _D), jnp.float32),
    )(ids, g2, pos_W, pos_b.reshape(1, _D))


def kernel(input_ids, token_table, pos_W, pos_b):
    b, seq = input_ids.shape
    ids = input_ids.astype(jnp.int32)
    idsf = ids.reshape(b * seq)
    g2 = _make_sc_gather(b * seq)(idsf, token_table)
    return _finalize(ids, g2, pos_W, pos_b)
